# Initial kernel scaffold; baseline (speedup 1.0000x reference)
import jax
import jax.numpy as jnp
from jax.experimental import pallas as pl


def kernel(src, tgt):
    raise NotImplementedError("write your pallas kernel here")


# speedup vs baseline: 17.2486x; 17.2486x over previous
import jax
import jax.numpy as jnp
from jax import lax
from jax.experimental import pallas as pl

_UP = 10
_K = 5
_STDF = 10.0
_CW = 8
_QT = 1024

_HI = lax.Precision.HIGHEST


def _std_body(tgt_ref, tgtT_ref, std_ref):
    tP = tgt_ref[0]
    tT = tgtT_ref[0]
    n = tP.shape[0]
    p2r = jnp.sum(tP * tP, axis=1, keepdims=True)
    p2c = jnp.sum(tT * tT, axis=0, keepdims=True)
    dot = lax.dot_general(tP, tT, (((1,), (0,)), ((), ())),
                          precision=_HI, preferred_element_type=jnp.float32)
    d2 = jnp.maximum(p2r + p2c - 2.0 * dot, 0.0)
    row = lax.broadcasted_iota(jnp.int32, (n, n), 0)
    m1 = jnp.min(d2, axis=0, keepdims=True)
    i1 = jnp.min(jnp.where(d2 == m1, row, n), axis=0, keepdims=True)
    d2b = jnp.where(row == i1, jnp.inf, d2)
    m2 = jnp.min(d2b, axis=0, keepdims=True)
    std_ref[0] = jnp.sqrt(m2 + 1e-10) * _STDF


def _loss_body(q_ref, tgtT_ref, srcT_ref, tgt_ref, src_ref, out_ref):
    q = q_ref[0]
    tT = tgtT_ref[0]
    sT = srcT_ref[0]
    tP = tgt_ref[0]
    sP = src_ref[0]
    t = q.shape[0]
    n = tT.shape[1]
    q2 = jnp.sum(q * q, axis=1, keepdims=True)
    col = lax.broadcasted_iota(jnp.int32, (t, n), 1)

    def top5(pT):
        p2 = jnp.sum(pT * pT, axis=0, keepdims=True)
        dot = lax.dot_general(q, pT, (((1,), (0,)), ((), ())),
                              precision=_HI, preferred_element_type=jnp.float32)
        d = jnp.maximum(q2 + p2 - 2.0 * dot, 0.0)
        vals, idxs = [], []
        for _ in range(_K):
            m = jnp.min(d, axis=1, keepdims=True)
            i = jnp.min(jnp.where(d == m, col, n), axis=1, keepdims=True)
            vals.append(m)
            idxs.append(i)
            d = jnp.where(col == i, jnp.inf, d)
        return vals, idxs

    vt, it = top5(tT)
    vs, isx = top5(sT)

    es = [jnp.exp(vt[0] - v) for v in vt]
    ssum = es[0] + es[1] + es[2] + es[3] + es[4]
    ws = [e / ssum for e in es]
    wsum = ws[0] + ws[1] + ws[2] + ws[3] + ws[4]

    udf_t = sum(w * jnp.sqrt(v + 1e-10) for w, v in zip(ws, vt))
    udf_s = sum(w * jnp.sqrt(v + 1e-10) for w, v in zip(ws, vs))

    wt = sum(jnp.where(col == i, w, 0.0) for w, i in zip(ws, it))
    wsrc = sum(jnp.where(col == i, w, 0.0) for w, i in zip(ws, isx))
    nnt = lax.dot_general(wt, tP, (((1,), (0,)), ((), ())),
                          precision=_HI, preferred_element_type=jnp.float32)
    nns = lax.dot_general(wsrc, sP, (((1,), (0,)), ((), ())),
                          precision=_HI, preferred_element_type=jnp.float32)
    gt = q * wsum - nnt
    gs = q * wsum - nns
    gerr = jnp.sum(jnp.abs(gs - gt), axis=1, keepdims=True)
    uerr = jnp.abs(udf_t - udf_s)
    err = uerr + gerr
    out_ref[0] = err * jnp.exp(-3.0 * err)


def kernel(src, tgt):
    b, n, _ = tgt.shape
    nq = n * (_UP + 1)
    padw = ((0, 0), (0, 0), (0, _CW - 3))
    tgt_p = jnp.pad(tgt, padw)
    src_p = jnp.pad(src, padw)
    tgt_t = jnp.swapaxes(tgt_p, 1, 2)
    src_t = jnp.swapaxes(src_p, 1, 2)

    std = pl.pallas_call(
        _std_body,
        grid=(b,),
        in_specs=[pl.BlockSpec((1, n, _CW), lambda i: (i, 0, 0)),
                  pl.BlockSpec((1, _CW, n), lambda i: (i, 0, 0))],
        out_specs=pl.BlockSpec((1, 1, n), lambda i: (i, 0, 0)),
        out_shape=jax.ShapeDtypeStruct((b, 1, n), jnp.float32),
    )(tgt_p, tgt_t)
    std = std.reshape(b, n, 1)

    noise = jax.random.normal(jax.random.key(42), (b, n, _UP, 3),
                              dtype=jnp.float32) * std[:, :, :, None]
    qgen = (tgt[:, :, None, :] + noise).reshape(b, -1, 3)
    query = jnp.concatenate([qgen, src], axis=1)
    q_p = jnp.pad(query, padw)

    nt = nq // _QT
    contrib = pl.pallas_call(
        _loss_body,
        grid=(b, nt),
        in_specs=[pl.BlockSpec((1, _QT, _CW), lambda i, j: (i, j, 0)),
                  pl.BlockSpec((1, _CW, n), lambda i, j: (i, 0, 0)),
                  pl.BlockSpec((1, _CW, n), lambda i, j: (i, 0, 0)),
                  pl.BlockSpec((1, n, _CW), lambda i, j: (i, 0, 0)),
                  pl.BlockSpec((1, n, _CW), lambda i, j: (i, 0, 0))],
        out_specs=pl.BlockSpec((1, _QT, 1), lambda i, j: (i, j, 0)),
        out_shape=jax.ShapeDtypeStruct((b, nq, 1), jnp.float32),
    )(q_p, tgt_t, src_t, tgt_p, src_p)
    return jnp.sum(contrib) / b / nq


# speedup vs baseline: 17.6577x; 1.0237x over previous
import jax
import jax.numpy as jnp
from jax import lax
from jax.experimental import pallas as pl

_UP = 10
_K = 5
_STDF = 10.0
_CW = 8
_QT = 1024

_HI = lax.Precision.HIGHEST
_IDXMASK = 1023
_VALMASK = -1024
_I32MAX = 2**31 - 1


def _std_body(tgt_ref, tgtT_ref, std_ref):
    tP = tgt_ref[0]
    tT = tgtT_ref[0]
    n = tP.shape[0]
    p2r = jnp.sum(tP * tP, axis=1, keepdims=True)
    p2c = jnp.sum(tT * tT, axis=0, keepdims=True)
    dot = lax.dot_general(tP, tT, (((1,), (0,)), ((), ())),
                          precision=_HI, preferred_element_type=jnp.float32)
    d2 = jnp.maximum(p2r + p2c - 2.0 * dot, 0.0)
    row = lax.broadcasted_iota(jnp.int32, (n, n), 0)
    m1 = jnp.min(d2, axis=0, keepdims=True)
    i1 = jnp.min(jnp.where(d2 == m1, row, n), axis=0, keepdims=True)
    d2b = jnp.where(row == i1, jnp.inf, d2)
    m2 = jnp.min(d2b, axis=0, keepdims=True)
    std_ref[0] = jnp.sqrt(m2 + 1e-10) * _STDF


def _loss_body(qa_ref, tgtBT_ref, srcBT_ref, tgt_ref, src_ref, out_ref):
    qa = qa_ref[0]
    tBT = tgtBT_ref[0]
    sBT = srcBT_ref[0]
    tP = tgt_ref[0]
    sP = src_ref[0]
    t = qa.shape[0]
    n = tBT.shape[1]
    col = lax.broadcasted_iota(jnp.int32, (t, n), 1)

    def top5(pBT):
        d2 = lax.dot_general(qa, pBT, (((1,), (0,)), ((), ())),
                             precision=_HI, preferred_element_type=jnp.float32)
        d2 = jnp.maximum(d2, 0.0)
        key = (lax.bitcast_convert_type(d2, jnp.int32) & _VALMASK) | col
        vals, idxs = [], []
        for _ in range(_K):
            m = jnp.min(key, axis=1, keepdims=True)
            key = jnp.where(key == m, _I32MAX, key)
            vals.append(lax.bitcast_convert_type(m & _VALMASK, jnp.float32))
            idxs.append(m & _IDXMASK)
        return vals, idxs

    vt, it = top5(tBT)
    vs, isx = top5(sBT)

    es = [jnp.exp(vt[0] - v) for v in vt]
    ssum = es[0] + es[1] + es[2] + es[3] + es[4]
    ws = [e / ssum for e in es]

    udf_t = sum(w * jnp.sqrt(v + 1e-10) for w, v in zip(ws, vt))
    udf_s = sum(w * jnp.sqrt(v + 1e-10) for w, v in zip(ws, vs))

    wt = sum(jnp.where(col == i, w, 0.0) for w, i in zip(ws, it))
    wsrc = sum(jnp.where(col == i, w, 0.0) for w, i in zip(ws, isx))
    nnt = lax.dot_general(wt, tP, (((1,), (0,)), ((), ())),
                          precision=_HI, preferred_element_type=jnp.float32)
    nns = lax.dot_general(wsrc, sP, (((1,), (0,)), ((), ())),
                          precision=_HI, preferred_element_type=jnp.float32)
    diff = jnp.abs(nnt - nns)
    ones = jnp.ones((_CW, 1), jnp.float32)
    gerr = lax.dot_general(diff, ones, (((1,), (0,)), ((), ())),
                           precision=_HI, preferred_element_type=jnp.float32)
    uerr = jnp.abs(udf_t - udf_s)
    err = uerr + gerr
    out_ref[0] = err * jnp.exp(-3.0 * err)


def _augment(pts):
    n2 = jnp.sum(pts * pts, axis=-1, keepdims=True)
    one = jnp.ones_like(n2)
    zero = jnp.zeros(pts.shape[:-1] + (_CW - 5,), pts.dtype)
    return jnp.concatenate([pts, n2, one, zero], axis=-1)


def kernel(src, tgt):
    b, n, _ = tgt.shape
    nq = n * (_UP + 1)
    padw = ((0, 0), (0, 0), (0, _CW - 3))
    tgt_p = jnp.pad(tgt, padw)
    src_p = jnp.pad(src, padw)
    tgt_t = jnp.swapaxes(tgt_p, 1, 2)

    std = pl.pallas_call(
        _std_body,
        grid=(b,),
        in_specs=[pl.BlockSpec((1, n, _CW), lambda i: (i, 0, 0)),
                  pl.BlockSpec((1, _CW, n), lambda i: (i, 0, 0))],
        out_specs=pl.BlockSpec((1, 1, n), lambda i: (i, 0, 0)),
        out_shape=jax.ShapeDtypeStruct((b, 1, n), jnp.float32),
    )(tgt_p, tgt_t)
    std = std.reshape(b, n, 1)

    noise = jax.random.normal(jax.random.key(42), (b, n, _UP, 3),
                              dtype=jnp.float32) * std[:, :, :, None]
    qgen = (tgt[:, :, None, :] + noise).reshape(b, -1, 3)
    query = jnp.concatenate([qgen, src], axis=1)

    qa = _augment(query)
    def _bmat(pts):
        n2 = jnp.sum(pts * pts, axis=-1, keepdims=True)
        one = jnp.ones_like(n2)
        zero = jnp.zeros(pts.shape[:-1] + (_CW - 5,), pts.dtype)
        return jnp.swapaxes(
            jnp.concatenate([-2.0 * pts, one, n2, zero], axis=-1), 1, 2)

    tgt_bt = _bmat(tgt)
    src_bt = _bmat(src)

    nt = nq // _QT
    contrib = pl.pallas_call(
        _loss_body,
        grid=(b, nt),
        in_specs=[pl.BlockSpec((1, _QT, _CW), lambda i, j: (i, j, 0)),
                  pl.BlockSpec((1, _CW, n), lambda i, j: (i, 0, 0)),
                  pl.BlockSpec((1, _CW, n), lambda i, j: (i, 0, 0)),
                  pl.BlockSpec((1, n, _CW), lambda i, j: (i, 0, 0)),
                  pl.BlockSpec((1, n, _CW), lambda i, j: (i, 0, 0))],
        out_specs=pl.BlockSpec((1, _QT, 1), lambda i, j: (i, j, 0)),
        out_shape=jax.ShapeDtypeStruct((b, nq, 1), jnp.float32),
    )(qa, tgt_bt, src_bt, tgt_p, src_p)
    return jnp.sum(contrib) / b / nq


# speedup vs baseline: 26.2867x; 1.4887x over previous
import jax
import jax.numpy as jnp
from jax import lax
from jax.experimental import pallas as pl
from jax.experimental.pallas import tpu as pltpu

_UP = 10
_K = 5
_STDF = 10.0
_CW = 8
_HI = lax.Precision.HIGHEST
_IDXMASK = 1023
_VALMASK = -1024
_I32MAX = 2**31 - 1


def _body(tgtT_ref, srcT_ref, tgt_ref, src_ref, t2_ref, s2_ref, noiseT_ref,
          out_ref, std_ref):
    j = pl.program_id(1)
    tT = tgtT_ref[0]
    sT = srcT_ref[0]
    tP = tgt_ref[0]
    sP = src_ref[0]
    t2 = t2_ref[0]
    s2 = s2_ref[0]
    n = tT.shape[1]
    row = lax.broadcasted_iota(jnp.int32, (n, n), 0)

    @pl.when(j == 0)
    def _():
        dot = lax.dot_general(tP, tT, (((1,), (0,)), ((), ())),
                              precision=_HI,
                              preferred_element_type=jnp.float32)
        t2c = jnp.sum(tT * tT, axis=0, keepdims=True)
        d2 = jnp.maximum(t2 + t2c - 2.0 * dot, 0.0)
        m1 = jnp.min(d2, axis=0, keepdims=True)
        i1 = jnp.min(jnp.where(d2 == m1, row, n), axis=0, keepdims=True)
        m2 = jnp.min(jnp.where(row == i1, jnp.inf, d2), axis=0,
                     keepdims=True)
        std_ref[...] = jnp.sqrt(m2 + 1e-10) * _STDF

    qT = jnp.where(j == _UP, sT, tT + noiseT_ref[0, 0] * std_ref[...])
    q2 = jnp.sum(qT * qT, axis=0, keepdims=True)

    def top5(pT, p2):
        dot = lax.dot_general(pT, qT, (((1,), (0,)), ((), ())),
                              precision=_HI,
                              preferred_element_type=jnp.float32)
        d2 = jnp.maximum(p2 + q2 - 2.0 * dot, 0.0)
        key = (lax.bitcast_convert_type(d2, jnp.int32) & _VALMASK) | row
        ms = []
        for k in range(_K):
            m = jnp.min(key, axis=0, keepdims=True)
            ms.append(m)
            if k + 1 < _K:
                key = jnp.where(key == m, _I32MAX, key)
        vals = [lax.bitcast_convert_type(m & _VALMASK, jnp.float32)
                for m in ms]
        idxs = [m & _IDXMASK for m in ms]
        return vals, idxs

    vt, it = top5(tP, t2)
    vs, isx = top5(sP, s2)

    es = [jnp.exp(vt[0] - v) for v in vt]
    ssum = es[0] + es[1] + es[2] + es[3] + es[4]
    ws = [e / ssum for e in es]

    udf_t = sum(w * jnp.sqrt(v + 1e-10) for w, v in zip(ws, vt))
    udf_s = sum(w * jnp.sqrt(v + 1e-10) for w, v in zip(ws, vs))

    def onehot(idxs):
        acc = 0.0
        for w, i in zip(reversed(ws), reversed(idxs)):
            acc = jnp.where(row == i, w, acc)
        return acc

    nnt = lax.dot_general(tT, onehot(it), (((1,), (0,)), ((), ())),
                          precision=_HI, preferred_element_type=jnp.float32)
    nns = lax.dot_general(sT, onehot(isx), (((1,), (0,)), ((), ())),
                          precision=_HI, preferred_element_type=jnp.float32)
    gerr = jnp.sum(jnp.abs(nnt - nns), axis=0, keepdims=True)
    err = jnp.abs(udf_t - udf_s) + gerr
    out_ref[0, 0] = err * jnp.exp(-3.0 * err)


def kernel(src, tgt):
    b, n, _ = tgt.shape
    padw = ((0, 0), (0, 0), (0, _CW - 3))
    tgt_p = jnp.pad(tgt, padw)
    src_p = jnp.pad(src, padw)
    tgt_t = jnp.swapaxes(tgt_p, 1, 2)
    src_t = jnp.swapaxes(src_p, 1, 2)
    t2 = jnp.sum(tgt * tgt, axis=-1, keepdims=True)
    s2 = jnp.sum(src * src, axis=-1, keepdims=True)
    noise = jax.random.normal(jax.random.key(42), (b, n, _UP, 3),
                              dtype=jnp.float32)
    noise_t = jnp.pad(jnp.transpose(noise, (0, 2, 3, 1)),
                      ((0, 0), (0, 0), (0, _CW - 3), (0, 0)))

    nt = _UP + 1
    per_q = pl.pallas_call(
        _body,
        grid=(b, nt),
        in_specs=[
            pl.BlockSpec((1, _CW, n), lambda i, j: (i, 0, 0)),
            pl.BlockSpec((1, _CW, n), lambda i, j: (i, 0, 0)),
            pl.BlockSpec((1, n, _CW), lambda i, j: (i, 0, 0)),
            pl.BlockSpec((1, n, _CW), lambda i, j: (i, 0, 0)),
            pl.BlockSpec((1, n, 1), lambda i, j: (i, 0, 0)),
            pl.BlockSpec((1, n, 1), lambda i, j: (i, 0, 0)),
            pl.BlockSpec((1, 1, _CW, n),
                         lambda i, j: (i, jnp.minimum(j, _UP - 1), 0, 0)),
        ],
        out_specs=pl.BlockSpec((1, 1, 1, n), lambda i, j: (i, j, 0, 0)),
        out_shape=jax.ShapeDtypeStruct((b, nt, 1, n), jnp.float32),
        scratch_shapes=[pltpu.VMEM((1, n), jnp.float32)],
    )(tgt_t, src_t, tgt_p, src_p, t2, s2, noise_t)
    return jnp.sum(per_q) / b / (n * nt)
